# Initial kernel scaffold; baseline (speedup 1.0000x reference)
import functools

import jax
import jax.numpy as jnp
from jax import lax
from jax.experimental import pallas as pl
from jax.experimental.pallas import tpu as pltpu
from jax.experimental.pallas import tpu_sc as plsc

NSONGS = 100000
DIM = 64
BATCH = 1024
HIST = 200

NC = 2
NS = 16
NW = NC * NS
ROWS_PER_W = BATCH // NW

CHUNKS = ((0, 128), (128, 72))

_sc_mesh = plsc.VectorSubcoreMesh(core_axis_name="c", subcore_axis_name="s")


@functools.partial(
    pl.kernel,
    out_type=jax.ShapeDtypeStruct((BATCH, DIM), jnp.float32),
    mesh=_sc_mesh,
    scratch_types=[
        pltpu.VMEM((HIST,), jnp.int32),
        pltpu.VMEM((HIST, DIM), jnp.float32),
        pltpu.VMEM((DIM,), jnp.float32),
        pltpu.SemaphoreType.DMA,
    ],
)
def _pool_sc(xp1_hbm, table_hbm, h_hbm, idx_v, rows_v, acc_v, sem):
    wid = lax.axis_index("s") * NC + lax.axis_index("c")
    zero = jnp.zeros((16,), jnp.float32)

    def row_body(i, _):
        b = wid * ROWS_PER_W + i
        pltpu.sync_copy(xp1_hbm.at[b], idx_v)
        copies = [
            pltpu.async_copy(
                table_hbm.at[idx_v.at[pl.ds(off, n)]],
                rows_v.at[pl.ds(off, n)],
                sem,
            )
            for off, n in CHUNKS
        ]
        for cp in copies:
            cp.wait()

        def t_body(j, accs):
            a = list(accs)
            t0 = j * 8
            for u in range(8):
                for c in range(4):
                    a[c] = a[c] + rows_v[t0 + u, pl.ds(c * 16, 16)]
            return tuple(a)

        accs = lax.fori_loop(0, HIST // 8, t_body, (zero,) * 4)
        for c in range(4):
            acc_v[pl.ds(c * 16, 16)] = accs[c]
        pltpu.sync_copy(acc_v, h_hbm.at[b])
        return 0

    lax.fori_loop(0, ROWS_PER_W, row_body, 0)


BN = 1024


def _decode_body(h_ref, w_ref, b_ref, o_ref):
    o_ref[...] = (
        lax.dot_general(
            h_ref[...],
            w_ref[...],
            (((1,), (1,)), ((), ())),
            preferred_element_type=jnp.float32,
        )
        + b_ref[...]
    )


def kernel(x, emb_weight, inv_w, inv_b):
    xp1 = (x + 1).astype(jnp.int32)
    h = _pool_sc(xp1, emb_weight)
    nblk = pl.cdiv(NSONGS, BN)
    return pl.pallas_call(
        _decode_body,
        grid=(nblk,),
        in_specs=[
            pl.BlockSpec((BATCH, DIM), lambda j: (0, 0)),
            pl.BlockSpec((BN, DIM), lambda j: (j, 0)),
            pl.BlockSpec((1, BN), lambda j: (0, j)),
        ],
        out_specs=pl.BlockSpec((BATCH, BN), lambda j: (0, j)),
        out_shape=jax.ShapeDtypeStruct((BATCH, NSONGS), jnp.float32),
    )(h, inv_w, inv_b.reshape(1, NSONGS))


# speedup vs baseline: 1.3226x; 1.3226x over previous
import functools

import jax
import jax.numpy as jnp
from jax import lax
from jax.experimental import pallas as pl
from jax.experimental.pallas import tpu as pltpu
from jax.experimental.pallas import tpu_sc as plsc

NSONGS = 100000
DIM = 64
BATCH = 1024
HIST = 200

NC = 2
NS = 16
NW = NC * NS
ROWS_PER_W = BATCH // NW

CHUNKS = ((0, 128), (128, 72))

_sc_mesh = plsc.VectorSubcoreMesh(core_axis_name="c", subcore_axis_name="s")


@functools.partial(
    pl.kernel,
    out_type=jax.ShapeDtypeStruct((BATCH, DIM), jnp.float32),
    mesh=_sc_mesh,
    scratch_types=[
        pltpu.VMEM((HIST,), jnp.int32),
        pltpu.VMEM((HIST, DIM), jnp.float32),
        pltpu.VMEM((DIM,), jnp.float32),
        pltpu.SemaphoreType.DMA,
    ],
    compiler_params=pltpu.CompilerParams(use_tc_tiling_on_sc=False),
)
def _pool_sc(xp1_hbm, table_hbm, h_hbm, idx_v, rows_v, acc_v, sem):
    wid = lax.axis_index("s") * NC + lax.axis_index("c")
    zero = jnp.zeros((16,), jnp.float32)

    def row_body(i, _):
        b = wid * ROWS_PER_W + i
        pltpu.sync_copy(xp1_hbm.at[b], idx_v)
        copies = [
            pltpu.async_copy(
                table_hbm.at[idx_v.at[pl.ds(off, n)]],
                rows_v.at[pl.ds(off, n)],
                sem,
            )
            for off, n in CHUNKS
        ]
        for cp in copies:
            cp.wait()

        def t_body(j, accs):
            a = list(accs)
            t0 = j * 8
            for u in range(8):
                for c in range(4):
                    a[c] = a[c] + rows_v[t0 + u, pl.ds(c * 16, 16)]
            return tuple(a)

        accs = lax.fori_loop(0, HIST // 8, t_body, (zero,) * 4)
        for c in range(4):
            acc_v[pl.ds(c * 16, 16)] = accs[c]
        pltpu.sync_copy(acc_v, h_hbm.at[b])
        return 0

    lax.fori_loop(0, ROWS_PER_W, row_body, 0)


BN = 1024


def _decode_body(h_ref, w_ref, b_ref, o_ref):
    o_ref[...] = (
        lax.dot_general(
            h_ref[...],
            w_ref[...],
            (((1,), (1,)), ((), ())),
            preferred_element_type=jnp.float32,
        )
        + b_ref[...]
    )


def kernel(x, emb_weight, inv_w, inv_b):
    xp1 = (x + 1).astype(jnp.int32)
    h = _pool_sc(xp1, emb_weight)
    nblk = pl.cdiv(NSONGS, BN)
    return pl.pallas_call(
        _decode_body,
        grid=(nblk,),
        in_specs=[
            pl.BlockSpec((BATCH, DIM), lambda j: (0, 0)),
            pl.BlockSpec((BN, DIM), lambda j: (j, 0)),
            pl.BlockSpec((1, BN), lambda j: (0, j)),
        ],
        out_specs=pl.BlockSpec((BATCH, BN), lambda j: (0, j)),
        out_shape=jax.ShapeDtypeStruct((BATCH, NSONGS), jnp.float32),
    )(h, inv_w, inv_b.reshape(1, NSONGS))


# speedup vs baseline: 1.3653x; 1.0323x over previous
import functools

import jax
import jax.numpy as jnp
from jax import lax
from jax.experimental import pallas as pl
from jax.experimental.pallas import tpu as pltpu
from jax.experimental.pallas import tpu_sc as plsc

NSONGS = 100000
DIM = 64
BATCH = 1024
HIST = 200

NC = 2
NS = 16
NW = NC * NS
ROWS_PER_W = BATCH // NW

CHUNKS = ((0, 128), (128, 72))

_sc_mesh = plsc.VectorSubcoreMesh(core_axis_name="c", subcore_axis_name="s")


@functools.partial(
    pl.kernel,
    out_type=jax.ShapeDtypeStruct((BATCH, DIM), jnp.float32),
    mesh=_sc_mesh,
    scratch_types=[
        pltpu.VMEM((HIST,), jnp.int32),
        pltpu.VMEM((HIST, DIM), jnp.float32),
        pltpu.VMEM((DIM,), jnp.float32),
        pltpu.SemaphoreType.DMA,
    ],
    compiler_params=pltpu.CompilerParams(use_tc_tiling_on_sc=False),
)
def _pool_sc(xp1_hbm, table_hbm, h_hbm, idx_v, rows_v, acc_v, sem):
    wid = lax.axis_index("s") * NC + lax.axis_index("c")
    zero = jnp.zeros((16,), jnp.float32)

    def row_body(i, _):
        b = wid * ROWS_PER_W + i
        pltpu.sync_copy(xp1_hbm.at[b], idx_v)
        copies = [
            pltpu.async_copy(
                table_hbm.at[idx_v.at[pl.ds(off, n)]],
                rows_v.at[pl.ds(off, n)],
                sem,
            )
            for off, n in CHUNKS
        ]
        for cp in copies:
            cp.wait()

        def t_body(j, accs):
            a = list(accs)
            t0 = j * 8
            for u in range(8):
                for c in range(4):
                    a[c] = a[c] + rows_v[t0 + u, pl.ds(c * 16, 16)]
            return tuple(a)

        accs = lax.fori_loop(0, HIST // 8, t_body, (zero,) * 4)
        for c in range(4):
            acc_v[pl.ds(c * 16, 16)] = accs[c]
        pltpu.sync_copy(acc_v, h_hbm.at[b])
        return 0

    lax.fori_loop(0, ROWS_PER_W, row_body, 0)


BN = 2048


def _decode_body(h_ref, w_ref, b_ref, o_ref):
    o_ref[...] = (
        lax.dot_general(
            h_ref[...],
            w_ref[...],
            (((1,), (1,)), ((), ())),
            preferred_element_type=jnp.float32,
        )
        + b_ref[...]
    )


def kernel(x, emb_weight, inv_w, inv_b):
    xp1 = (x + 1).astype(jnp.int32)
    h = _pool_sc(xp1, emb_weight)
    nblk = pl.cdiv(NSONGS, BN)
    return pl.pallas_call(
        _decode_body,
        grid=(nblk,),
        in_specs=[
            pl.BlockSpec((BATCH, DIM), lambda j: (0, 0)),
            pl.BlockSpec((BN, DIM), lambda j: (j, 0)),
            pl.BlockSpec((1, BN), lambda j: (0, j)),
        ],
        out_specs=pl.BlockSpec((BATCH, BN), lambda j: (0, j)),
        out_shape=jax.ShapeDtypeStruct((BATCH, NSONGS), jnp.float32),
    )(h, inv_w, inv_b.reshape(1, NSONGS))


# speedup vs baseline: 1.3704x; 1.0038x over previous
import functools

import jax
import jax.numpy as jnp
from jax import lax
from jax.experimental import pallas as pl
from jax.experimental.pallas import tpu as pltpu
from jax.experimental.pallas import tpu_sc as plsc

NSONGS = 100000
DIM = 64
BATCH = 1024
HIST = 200

NC = 2
NS = 16
NW = NC * NS
ROWS_PER_W = BATCH // NW

CHUNKS = ((0, 128), (128, 72))

_sc_mesh = plsc.VectorSubcoreMesh(core_axis_name="c", subcore_axis_name="s")


@functools.partial(
    pl.kernel,
    out_type=jax.ShapeDtypeStruct((BATCH, DIM), jnp.float32),
    mesh=_sc_mesh,
    scratch_types=[
        pltpu.VMEM((HIST,), jnp.int32),
        pltpu.VMEM((HIST, DIM), jnp.float32),
        pltpu.VMEM((DIM,), jnp.float32),
        pltpu.SemaphoreType.DMA,
    ],
    compiler_params=pltpu.CompilerParams(use_tc_tiling_on_sc=False),
)
def _pool_sc(xp1_hbm, table_hbm, h_hbm, idx_v, rows_v, acc_v, sem):
    wid = lax.axis_index("s") * NC + lax.axis_index("c")
    zero = jnp.zeros((16,), jnp.float32)

    def row_body(i, _):
        b = wid * ROWS_PER_W + i
        pltpu.sync_copy(xp1_hbm.at[b], idx_v)
        copies = [
            pltpu.async_copy(
                table_hbm.at[idx_v.at[pl.ds(off, n)]],
                rows_v.at[pl.ds(off, n)],
                sem,
            )
            for off, n in CHUNKS
        ]
        for cp in copies:
            cp.wait()

        def t_body(j, accs):
            a = list(accs)
            t0 = j * 8
            for u in range(8):
                for c in range(4):
                    a[c] = a[c] + rows_v[t0 + u, pl.ds(c * 16, 16)]
            return tuple(a)

        accs = lax.fori_loop(0, HIST // 8, t_body, (zero,) * 4)
        for c in range(4):
            acc_v[pl.ds(c * 16, 16)] = accs[c]
        pltpu.sync_copy(acc_v, h_hbm.at[b])
        return 0

    lax.fori_loop(0, ROWS_PER_W, row_body, 0)


BN = 4096


def _decode_body(h_ref, w_ref, b_ref, o_ref):
    o_ref[...] = (
        lax.dot_general(
            h_ref[...],
            w_ref[...],
            (((1,), (1,)), ((), ())),
            preferred_element_type=jnp.float32,
        )
        + b_ref[...]
    )


def kernel(x, emb_weight, inv_w, inv_b):
    xp1 = (x + 1).astype(jnp.int32)
    h = _pool_sc(xp1, emb_weight)
    nblk = pl.cdiv(NSONGS, BN)
    return pl.pallas_call(
        _decode_body,
        grid=(nblk,),
        in_specs=[
            pl.BlockSpec((BATCH, DIM), lambda j: (0, 0)),
            pl.BlockSpec((BN, DIM), lambda j: (j, 0)),
            pl.BlockSpec((1, BN), lambda j: (0, j)),
        ],
        out_specs=pl.BlockSpec((BATCH, BN), lambda j: (0, j)),
        out_shape=jax.ShapeDtypeStruct((BATCH, NSONGS), jnp.float32),
    )(h, inv_w, inv_b.reshape(1, NSONGS))


# speedup vs baseline: 1.4329x; 1.0456x over previous
import functools

import jax
import jax.numpy as jnp
from jax import lax
from jax.experimental import pallas as pl
from jax.experimental.pallas import tpu as pltpu
from jax.experimental.pallas import tpu_sc as plsc

NSONGS = 100000
DIM = 64
BATCH = 1024
HIST = 200
HIST_PAD = 208

NC = 2
NS = 16
NW = NC * NS
ROWS_PER_W = BATCH // NW

CHUNKS = ((0, 128), (128, 72))

_sc_mesh = plsc.VectorSubcoreMesh(core_axis_name="c", subcore_axis_name="s")


@functools.partial(
    pl.kernel,
    out_type=jax.ShapeDtypeStruct((BATCH, DIM), jnp.float32),
    mesh=_sc_mesh,
    scratch_types=[
        pltpu.VMEM((2, HIST_PAD), jnp.int32),
        pltpu.VMEM((2, HIST, DIM), jnp.float32),
        pltpu.VMEM((ROWS_PER_W, DIM), jnp.float32),
        pltpu.SemaphoreType.DMA,
        pltpu.SemaphoreType.DMA,
    ],
    compiler_params=pltpu.CompilerParams(use_tc_tiling_on_sc=False),
)
def _pool_sc(x_hbm, table_hbm, h_hbm, idx_v, rows_v, hbuf_v, sem0, sem1):
    wid = lax.axis_index("s") * NC + lax.axis_index("c")
    base = wid * ROWS_PER_W
    zero = jnp.zeros((16,), jnp.float32)
    sems = (sem0, sem1)
    ione = jnp.ones((16,), jnp.int32)
    izero = jnp.zeros((16,), jnp.int32)

    def fetch_issue(row, buf):
        ib = idx_v.at[buf]
        ib[pl.ds(192, 16)] = izero
        pltpu.sync_copy(x_hbm.at[row], ib.at[pl.ds(0, HIST)])
        for c in range(HIST_PAD // 16):
            ib[pl.ds(c * 16, 16)] = ib[pl.ds(c * 16, 16)] + ione
        for off, n in CHUNKS:
            pltpu.async_copy(
                table_hbm.at[ib.at[pl.ds(off, n)]],
                rows_v.at[buf].at[pl.ds(off, n)],
                sems[buf],
            )

    def drain(buf):
        for off, n in CHUNKS:
            pltpu.make_async_copy(
                table_hbm.at[idx_v.at[buf].at[pl.ds(off, n)]],
                rows_v.at[buf].at[pl.ds(off, n)],
                sems[buf],
            ).wait()

    def reduce_into(local_row, buf):
        rb = rows_v.at[buf]

        def t_body(j, accs):
            a = list(accs)
            t0 = j * 8
            for u in range(8):
                for c in range(4):
                    a[c] = a[c] + rb[t0 + u, pl.ds(c * 16, 16)]
            return tuple(a)

        accs = lax.fori_loop(0, HIST // 8, t_body, (zero,) * 4)
        for c in range(4):
            hbuf_v[local_row, pl.ds(c * 16, 16)] = accs[c]

    fetch_issue(base, 0)

    def g_body(g, _):
        fetch_issue(base + 2 * g + 1, 1)
        drain(0)
        reduce_into(2 * g, 0)

        @pl.when(g < ROWS_PER_W // 2 - 1)
        def _():
            fetch_issue(base + 2 * g + 2, 0)

        drain(1)
        reduce_into(2 * g + 1, 1)
        return 0

    lax.fori_loop(0, ROWS_PER_W // 2, g_body, 0)
    pltpu.sync_copy(hbuf_v, h_hbm.at[pl.ds(base, ROWS_PER_W)])


BN = 4096


def _decode_body(h_ref, w_ref, b_ref, o_ref):
    o_ref[...] = (
        lax.dot_general(
            h_ref[...],
            w_ref[...],
            (((1,), (1,)), ((), ())),
            preferred_element_type=jnp.float32,
        )
        + b_ref[...]
    )


def kernel(x, emb_weight, inv_w, inv_b):
    h = _pool_sc(x.astype(jnp.int32), emb_weight)
    nblk = pl.cdiv(NSONGS, BN)
    return pl.pallas_call(
        _decode_body,
        grid=(nblk,),
        in_specs=[
            pl.BlockSpec((BATCH, DIM), lambda j: (0, 0)),
            pl.BlockSpec((BN, DIM), lambda j: (j, 0)),
            pl.BlockSpec((1, BN), lambda j: (0, j)),
        ],
        out_specs=pl.BlockSpec((BATCH, BN), lambda j: (0, j)),
        out_shape=jax.ShapeDtypeStruct((BATCH, NSONGS), jnp.float32),
    )(h, inv_w, inv_b.reshape(1, NSONGS))


# speedup vs baseline: 1.4439x; 1.0077x over previous
import functools

import jax
import jax.numpy as jnp
from jax import lax
from jax.experimental import pallas as pl
from jax.experimental.pallas import tpu as pltpu
from jax.experimental.pallas import tpu_sc as plsc

NSONGS = 100000
DIM = 64
BATCH = 1024
HIST = 200
HIST_PAD = 208

NC = 2
NS = 16
NW = NC * NS
ROWS_PER_W = BATCH // NW

CHUNKS = ((0, 128), (128, 72))

_sc_mesh = plsc.VectorSubcoreMesh(core_axis_name="c", subcore_axis_name="s")


IDX_PER_W = ROWS_PER_W * HIST


@functools.partial(
    pl.kernel,
    out_type=jax.ShapeDtypeStruct((BATCH, DIM), jnp.float32),
    mesh=_sc_mesh,
    scratch_types=[
        pltpu.VMEM((IDX_PER_W,), jnp.int32),
        pltpu.VMEM((2, HIST, DIM), jnp.float32),
        pltpu.VMEM((ROWS_PER_W, DIM), jnp.float32),
        pltpu.SemaphoreType.DMA,
        pltpu.SemaphoreType.DMA,
    ],
    compiler_params=pltpu.CompilerParams(use_tc_tiling_on_sc=False),
)
def _pool_sc(x_hbm, table_hbm, h_hbm, idx_v, rows_v, hbuf_v, sem0, sem1):
    wid = lax.axis_index("s") * NC + lax.axis_index("c")
    base = wid * ROWS_PER_W
    zero = jnp.zeros((16,), jnp.float32)
    sems = (sem0, sem1)
    ione = jnp.ones((16,), jnp.int32)

    pltpu.sync_copy(x_hbm.at[pl.ds(wid * IDX_PER_W, IDX_PER_W)], idx_v)

    def inc_body(g, _):
        for u in range(8):
            off = g * 128 + u * 16
            idx_v[pl.ds(off, 16)] = idx_v[pl.ds(off, 16)] + ione
        return 0

    lax.fori_loop(0, IDX_PER_W // 128, inc_body, 0)

    def fetch_issue(local_row, buf):
        for off, n in CHUNKS:
            pltpu.async_copy(
                table_hbm.at[idx_v.at[pl.ds(local_row * HIST + off, n)]],
                rows_v.at[buf].at[pl.ds(off, n)],
                sems[buf],
            )

    def drain(local_row, buf):
        for off, n in CHUNKS:
            pltpu.make_async_copy(
                table_hbm.at[idx_v.at[pl.ds(local_row * HIST + off, n)]],
                rows_v.at[buf].at[pl.ds(off, n)],
                sems[buf],
            ).wait()

    def reduce_into(local_row, buf):
        rb = rows_v.at[buf]

        def t_body(j, accs):
            a = list(accs)
            t0 = j * 8
            for u in range(8):
                for c in range(4):
                    a[c] = a[c] + rb[t0 + u, pl.ds(c * 16, 16)]
            return tuple(a)

        accs = lax.fori_loop(0, HIST // 8, t_body, (zero,) * 4)
        for c in range(4):
            hbuf_v[local_row, pl.ds(c * 16, 16)] = accs[c]

    fetch_issue(0, 0)

    def g_body(g, _):
        fetch_issue(2 * g + 1, 1)
        drain(2 * g, 0)
        reduce_into(2 * g, 0)

        @pl.when(g < ROWS_PER_W // 2 - 1)
        def _():
            fetch_issue(2 * g + 2, 0)

        drain(2 * g + 1, 1)
        reduce_into(2 * g + 1, 1)
        return 0

    lax.fori_loop(0, ROWS_PER_W // 2, g_body, 0)
    pltpu.sync_copy(hbuf_v, h_hbm.at[pl.ds(base, ROWS_PER_W)])


BN = 4096


def _decode_body(h_ref, w_ref, b_ref, o_ref):
    o_ref[...] = (
        lax.dot_general(
            h_ref[...],
            w_ref[...],
            (((1,), (1,)), ((), ())),
            preferred_element_type=jnp.float32,
        )
        + b_ref[...]
    )


def kernel(x, emb_weight, inv_w, inv_b):
    h = _pool_sc(x.astype(jnp.int32).reshape(BATCH * HIST), emb_weight)
    nblk = pl.cdiv(NSONGS, BN)
    return pl.pallas_call(
        _decode_body,
        grid=(nblk,),
        in_specs=[
            pl.BlockSpec((BATCH, DIM), lambda j: (0, 0)),
            pl.BlockSpec((BN, DIM), lambda j: (j, 0)),
            pl.BlockSpec((1, BN), lambda j: (0, j)),
        ],
        out_specs=pl.BlockSpec((BATCH, BN), lambda j: (0, j)),
        out_shape=jax.ShapeDtypeStruct((BATCH, NSONGS), jnp.float32),
    )(h, inv_w, inv_b.reshape(1, NSONGS))
